# Initial kernel scaffold; baseline (speedup 1.0000x reference)
import jax
import jax.numpy as jnp
from jax.experimental import pallas as pl


def kernel(normu, cls):
    raise NotImplementedError("write your pallas kernel here")


# speedup vs baseline: 17.3507x; 17.3507x over previous
import jax
import jax.numpy as jnp
from jax.experimental import pallas as pl
from jax.experimental.pallas import tpu as pltpu

_N = 64
_C = 100000
_K = 8
_INV_T = 0.5
_NB = 8
_BLK = 12800
_NEG = float("-inf")
_IMAX = 2**31 - 1


def _stage1(x_ref, outv_ref, outi_ref, sum_ref, runv_ref, runi_ref):
    b = pl.program_id(0)

    @pl.when(b == 0)
    def _init():
        sum_ref[...] = jnp.zeros_like(sum_ref)
        runv_ref[...] = jnp.full((_N, _K), _NEG, jnp.float32)
        runi_ref[...] = jnp.full((_N, _K), _IMAX, jnp.int32)

    x = x_ref[...]
    col = jax.lax.broadcasted_iota(jnp.int32, (_N, _BLK), 1) + b * _BLK
    x = jnp.where(col < _C, x, _NEG)
    e = jnp.exp(x * _INV_T)
    sum_ref[...] += jnp.sum(e, axis=1, keepdims=True)

    bvs, bis = [], []
    for i in range(_K):
        m = jnp.max(x, axis=1, keepdims=True)
        hit = x == m
        idx = jnp.min(jnp.where(hit, col, _IMAX), axis=1, keepdims=True)
        bvs.append(m)
        bis.append(idx)
        x = jnp.where(hit & (col == idx), _NEG, x)

    V = jnp.concatenate([runv_ref[...]] + bvs, axis=1)
    I = jnp.concatenate([runi_ref[...]] + bis, axis=1)
    nv, ni = [], []
    for i in range(_K):
        m = jnp.max(V, axis=1, keepdims=True)
        hit = V == m
        idx = jnp.min(jnp.where(hit, I, _IMAX), axis=1, keepdims=True)
        nv.append(m)
        ni.append(idx)
        V = jnp.where(hit & (I == idx), _NEG, V)
    runv_ref[...] = jnp.concatenate(nv, axis=1)
    runi_ref[...] = jnp.concatenate(ni, axis=1)

    @pl.when(b == _NB - 1)
    def _final():
        topv = runv_ref[...]
        denom = sum_ref[...]
        for i in range(_K):
            e = jnp.exp(topv[:, i:i + 1] * _INV_T)
            outv_ref[:, i:i + 1] = e / denom
            denom = denom - e
        outi_ref[...] = runi_ref[...]


def _stage2(outi_ref, outv_ref, o_ref):
    b = pl.program_id(0)
    col = jax.lax.broadcasted_iota(jnp.int32, (_N, _BLK), 1) + b * _BLK
    acc = jnp.zeros((_N, _BLK), jnp.float32)
    for i in range(_K):
        acc = jnp.where(col == outi_ref[:, i:i + 1], outv_ref[:, i:i + 1], acc)
    o_ref[...] = acc


def kernel(normu, cls):
    outv, outi = pl.pallas_call(
        _stage1,
        grid=(_NB,),
        in_specs=[pl.BlockSpec((_N, _BLK), lambda b: (0, b))],
        out_specs=[
            pl.BlockSpec((_N, _K), lambda b: (0, 0)),
            pl.BlockSpec((_N, _K), lambda b: (0, 0)),
        ],
        out_shape=[
            jax.ShapeDtypeStruct((_N, _K), jnp.float32),
            jax.ShapeDtypeStruct((_N, _K), jnp.int32),
        ],
        scratch_shapes=[
            pltpu.VMEM((_N, 1), jnp.float32),
            pltpu.VMEM((_N, _K), jnp.float32),
            pltpu.VMEM((_N, _K), jnp.int32),
        ],
    )(cls)

    classes = pl.pallas_call(
        _stage2,
        grid=(_NB,),
        in_specs=[
            pl.BlockSpec((_N, _K), lambda b: (0, 0)),
            pl.BlockSpec((_N, _K), lambda b: (0, 0)),
        ],
        out_specs=pl.BlockSpec((_N, _BLK), lambda b: (0, b)),
        out_shape=jax.ShapeDtypeStruct((_N, _C), jnp.float32),
    )(outi, outv)

    return (normu, classes)
